# Initial kernel scaffold; baseline (speedup 1.0000x reference)
import jax
import jax.numpy as jnp
from jax.experimental import pallas as pl


def kernel(input_ids, weight):
    raise NotImplementedError("write your pallas kernel here")


# speedup vs baseline: 4.5355x; 4.5355x over previous
import functools

import jax
import jax.numpy as jnp
from jax import lax
from jax.experimental import pallas as pl
from jax.experimental.pallas import tpu as pltpu
from jax.experimental.pallas import tpu_sc as plsc

BATCH = 4096
SEQ = 50
VOCAB = 100000
EMBED = 64

NC = 2
NS = 16
NW = NC * NS
B = BATCH * SEQ
B_PER_W = B // NW
CHUNK = 128
NCHUNK = B_PER_W // CHUNK


def _sc_gather(idx_grid, weight):
    mesh = plsc.VectorSubcoreMesh(core_axis_name="c", subcore_axis_name="s")

    @functools.partial(
        pl.kernel,
        mesh=mesh,
        out_type=jax.ShapeDtypeStruct((B, EMBED), jnp.float32),
        scratch_types=[
            pltpu.VMEM((NCHUNK, CHUNK), jnp.int32),
            pltpu.VMEM((2, CHUNK, EMBED), jnp.float32),
            pltpu.SemaphoreType.DMA,
            pltpu.SemaphoreType.DMA,
        ],
        compiler_params=pltpu.CompilerParams(use_tc_tiling_on_sc=False),
    )
    def k(idx_hbm, table_hbm, out_hbm, idx_v, rows_v, sem0, sem1):
        wid = lax.axis_index("s") * NC + lax.axis_index("c")
        base = wid * B_PER_W
        pltpu.sync_copy(idx_hbm.at[wid], idx_v)

        sems = (sem0, sem1)
        pltpu.async_copy(table_hbm.at[idx_v.at[0]], rows_v.at[0], sem0)

        def body(g, carry):
            for b in range(2):
                j = g * 2 + b
                nxt = (b + 1) % 2
                @pl.when(j + 1 < NCHUNK)
                def _():
                    pltpu.async_copy(
                        table_hbm.at[idx_v.at[j + 1]], rows_v.at[nxt], sems[nxt]
                    )
                pltpu.make_async_copy(
                    table_hbm.at[idx_v.at[j]], rows_v.at[b], sems[b]
                ).wait()
                pltpu.sync_copy(
                    rows_v.at[b], out_hbm.at[pl.ds(base + j * CHUNK, CHUNK)]
                )
            return carry

        lax.fori_loop(0, NCHUNK // 2, body, 0)

    return k(idx_grid, weight)


def kernel(input_ids, weight):
    idx_grid = input_ids.astype(jnp.int32).reshape(NW, NCHUNK, CHUNK)
    out = _sc_gather(idx_grid, weight)
    return out.reshape(BATCH, SEQ, EMBED)


# speedup vs baseline: 4.6856x; 1.0331x over previous
import functools

import jax
import jax.numpy as jnp
from jax import lax
from jax.experimental import pallas as pl
from jax.experimental.pallas import tpu as pltpu
from jax.experimental.pallas import tpu_sc as plsc

BATCH = 4096
SEQ = 50
VOCAB = 100000
EMBED = 64

NC = 2
NS = 16
NW = NC * NS
B = BATCH * SEQ
B_PER_W = B // NW
CHUNK = 128
NCHUNK = B_PER_W // CHUNK
NBUF = 5
PF = 3


def _sc_gather(idx_grid, weight):
    mesh = plsc.VectorSubcoreMesh(core_axis_name="c", subcore_axis_name="s")

    @functools.partial(
        pl.kernel,
        mesh=mesh,
        out_type=jax.ShapeDtypeStruct((B, EMBED), jnp.float32),
        scratch_types=[
            pltpu.VMEM((NCHUNK, CHUNK), jnp.int32),
            pltpu.VMEM((NBUF, CHUNK, EMBED), jnp.float32),
            [pltpu.SemaphoreType.DMA] * NBUF,
            [pltpu.SemaphoreType.DMA] * NBUF,
        ],
        compiler_params=pltpu.CompilerParams(use_tc_tiling_on_sc=False),
    )
    def k(idx_hbm, table_hbm, out_hbm, idx_v, rows_v, g_sems, o_sems):
        wid = lax.axis_index("s") * NC + lax.axis_index("c")
        base = wid * B_PER_W
        pltpu.sync_copy(idx_hbm.at[wid], idx_v)

        for j in range(PF):
            pltpu.async_copy(table_hbm.at[idx_v.at[j]], rows_v.at[j], g_sems[j])

        def body(g, carry):
            for b in range(NBUF):
                j = g * NBUF + b
                pltpu.make_async_copy(
                    table_hbm.at[idx_v.at[b]], rows_v.at[b], g_sems[b]
                ).wait()
                pltpu.async_copy(
                    rows_v.at[b],
                    out_hbm.at[pl.ds(base + j * CHUNK, CHUNK)],
                    o_sems[b],
                )
                bb = (b + PF) % NBUF

                @pl.when(jnp.logical_and(j + PF < NCHUNK, j + PF >= NBUF))
                def _():
                    pltpu.make_async_copy(
                        rows_v.at[bb],
                        out_hbm.at[pl.ds(base, CHUNK)],
                        o_sems[bb],
                    ).wait()

                @pl.when(j + PF < NCHUNK)
                def _():
                    pltpu.async_copy(
                        table_hbm.at[idx_v.at[j + PF]], rows_v.at[bb], g_sems[bb]
                    )
            return carry

        lax.fori_loop(0, NCHUNK // NBUF, body, 0)

        for b in range(NBUF):
            pltpu.make_async_copy(
                rows_v.at[b], out_hbm.at[pl.ds(base, CHUNK)], o_sems[b]
            ).wait()

    return k(idx_grid, weight)


def kernel(input_ids, weight):
    idx_grid = input_ids.astype(jnp.int32).reshape(NW, NCHUNK, CHUNK)
    out = _sc_gather(idx_grid, weight)
    return out.reshape(BATCH, SEQ, EMBED)
